# Initial kernel scaffold; baseline (speedup 1.0000x reference)
import jax
import jax.numpy as jnp
from jax.experimental import pallas as pl


def kernel(images, coordinates):
    raise NotImplementedError("write your pallas kernel here")


# speedup vs baseline: 1.0988x; 1.0988x over previous
import functools

import jax
import jax.numpy as jnp
from jax import lax
from jax.experimental import pallas as pl
from jax.experimental.pallas import tpu as pltpu
from jax.experimental.pallas import tpu_sc as plsc

H = 384
W = 384
C = 192
N = H * W
NC = 2
NS = 16
NW = NC * NS
LANES = 16
N_PER_W = N // NW
P = 64
NCHUNK = N_PER_W // P


def _make_sc_kernel():
    mesh = plsc.VectorSubcoreMesh(core_axis_name="c", subcore_axis_name="s")

    @functools.partial(
        pl.kernel,
        mesh=mesh,
        compiler_params=pltpu.CompilerParams(use_tc_tiling_on_sc=False),
        out_type=jax.ShapeDtypeStruct((N, C), jnp.float32),
        scratch_types=[
            pltpu.VMEM((P,), jnp.float32),
            pltpu.VMEM((P,), jnp.float32),
            pltpu.VMEM((P,), jnp.int32),
            pltpu.VMEM((P,), jnp.int32),
            pltpu.VMEM((P,), jnp.int32),
            pltpu.VMEM((P,), jnp.int32),
            pltpu.VMEM((P,), jnp.float32),
            pltpu.VMEM((P,), jnp.float32),
            pltpu.VMEM((P,), jnp.float32),
            pltpu.VMEM((P,), jnp.float32),
            pltpu.VMEM((P, C), jnp.float32),
            pltpu.VMEM((P, C), jnp.float32),
            pltpu.VMEM((P, C), jnp.float32),
            pltpu.VMEM((P, C), jnp.float32),
            pltpu.VMEM((P, C), jnp.float32),
            pltpu.SemaphoreType.DMA,
            pltpu.SemaphoreType.DMA,
            pltpu.SemaphoreType.DMA,
            pltpu.SemaphoreType.DMA,
        ],
    )
    def bilinear_sc(table_hbm, xs_hbm, ys_hbm, out_hbm,
                    xv, yv, ia, ib, ic, idd, war, wbr, wcr, wdr,
                    ar, br, cr, dr, ov, sa, sb, sc, sd):
        wid = lax.axis_index("s") * NC + lax.axis_index("c")
        base = wid * N_PER_W

        def chunk_body(k, carry):
            off = base + k * P
            pltpu.sync_copy(xs_hbm.at[pl.ds(off, P)], xv)
            pltpu.sync_copy(ys_hbm.at[pl.ds(off, P)], yv)
            for i in range(P // LANES):
                sl = pl.ds(i * LANES, LANES)
                x = xv[sl]
                y = yv[sl]
                x0i = jnp.minimum(jnp.maximum(x.astype(jnp.int32), 0), H - 1)
                y0i = jnp.minimum(jnp.maximum(y.astype(jnp.int32), 0), W - 1)
                x1i = jnp.minimum(x0i + 1, H - 1)
                y1i = jnp.minimum(y0i + 1, W - 1)
                xc = jnp.minimum(jnp.maximum(x, 0.0), float(H - 1))
                yc = jnp.minimum(jnp.maximum(y, 0.0), float(W - 1))
                x0f = x0i.astype(jnp.float32)
                x1f = x1i.astype(jnp.float32)
                y0f = y0i.astype(jnp.float32)
                y1f = y1i.astype(jnp.float32)
                ia[sl] = x0i * W + y0i
                ib[sl] = x0i * W + y1i
                ic[sl] = x1i * W + y0i
                idd[sl] = x1i * W + y1i
                war[sl] = (x1f - xc) * (y1f - yc)
                wbr[sl] = (x1f - xc) * (yc - y0f)
                wcr[sl] = (xc - x0f) * (y1f - yc)
                wdr[sl] = (xc - x0f) * (yc - y0f)
            ca = pltpu.async_copy(table_hbm.at[ia], ar, sa)
            cb = pltpu.async_copy(table_hbm.at[ib], br, sb)
            cc = pltpu.async_copy(table_hbm.at[ic], cr, sc)
            cd = pltpu.async_copy(table_hbm.at[idd], dr, sd)
            ca.wait()
            cb.wait()
            cc.wait()
            cd.wait()

            def grp_body(g, c2):
                gs = pl.ds(g * LANES, LANES)
                wa16 = war[gs]
                wb16 = wbr[gs]
                wc16 = wcr[gs]
                wd16 = wdr[gs]
                row0 = g * LANES
                for p in range(LANES):
                    wa = wa16[p]
                    wb = wb16[p]
                    wc = wc16[p]
                    wd = wd16[p]
                    r = row0 + p
                    for j in range(C // LANES):
                        sj = pl.ds(j * LANES, LANES)
                        ov[r, sj] = (wa * ar[r, sj] + wb * br[r, sj]
                                     + wc * cr[r, sj] + wd * dr[r, sj])
                return c2

            lax.fori_loop(0, P // LANES, grp_body, 0)
            pltpu.sync_copy(ov, out_hbm.at[pl.ds(off, P)])
            return carry

        lax.fori_loop(0, NCHUNK, chunk_body, 0)

    return bilinear_sc


_bilinear_sc = _make_sc_kernel()


def kernel(images, coordinates):
    B, c, h, w = images.shape
    table = images.reshape(c, h * w).T
    xs = coordinates[:, 0].reshape(-1)
    ys = coordinates[:, 1].reshape(-1)
    out_nc = _bilinear_sc(table, xs, ys)
    new_h, new_w = coordinates.shape[2], coordinates.shape[3]
    return out_nc.T.reshape(B, c, new_h, new_w)


# speedup vs baseline: 1.1126x; 1.0125x over previous
import functools

import jax
import jax.numpy as jnp
from jax import lax
from jax.experimental import pallas as pl
from jax.experimental.pallas import tpu as pltpu
from jax.experimental.pallas import tpu_sc as plsc

H = 384
W = 384
C = 192
N = H * W
NC = 2
NS = 16
NW = NC * NS
LANES = 16
N_PER_W = N // NW
P = 48
NCHUNK = N_PER_W // P


def _scratch_types():
    per_set = (
        [pltpu.VMEM((P,), jnp.float32)] * 2
        + [pltpu.VMEM((P,), jnp.int32)] * 4
        + [pltpu.VMEM((P,), jnp.float32)] * 4
        + [pltpu.VMEM((P, C), jnp.float32)] * 4
        + [pltpu.VMEM((P, C), jnp.float32)]
    )
    return (per_set * 2
            + [pltpu.SemaphoreType.DMA] * 8
            + [pltpu.SemaphoreType.DMA] * 2)


def _make_sc_kernel():
    mesh = plsc.VectorSubcoreMesh(core_axis_name="c", subcore_axis_name="s")

    @functools.partial(
        pl.kernel,
        mesh=mesh,
        compiler_params=pltpu.CompilerParams(use_tc_tiling_on_sc=False),
        out_type=jax.ShapeDtypeStruct((N, C), jnp.float32),
        scratch_types=_scratch_types(),
    )
    def bilinear_sc(table_hbm, xs_hbm, ys_hbm, out_hbm, *s):
        it = iter(s)
        xv, yv, idx, wts, rows, ov = [], [], [], [], [], []
        for _ in range(2):
            xv.append(next(it))
            yv.append(next(it))
            idx.append([next(it) for _ in range(4)])
            wts.append([next(it) for _ in range(4)])
            rows.append([next(it) for _ in range(4)])
            ov.append(next(it))
        sg = [[next(it) for _ in range(4)] for _ in range(2)]
        so = [next(it) for _ in range(2)]

        wid = lax.axis_index("s") * NC + lax.axis_index("c")
        base = wid * N_PER_W

        def fire(k, b):
            off = base + k * P
            pltpu.sync_copy(xs_hbm.at[pl.ds(off, P)], xv[b])
            pltpu.sync_copy(ys_hbm.at[pl.ds(off, P)], yv[b])
            for i in range(P // LANES):
                sl = pl.ds(i * LANES, LANES)
                x = xv[b][sl]
                y = yv[b][sl]
                x0i = jnp.minimum(jnp.maximum(x.astype(jnp.int32), 0), H - 1)
                y0i = jnp.minimum(jnp.maximum(y.astype(jnp.int32), 0), W - 1)
                x1i = jnp.minimum(x0i + 1, H - 1)
                y1i = jnp.minimum(y0i + 1, W - 1)
                xc = jnp.minimum(jnp.maximum(x, 0.0), float(H - 1))
                yc = jnp.minimum(jnp.maximum(y, 0.0), float(W - 1))
                x0f = x0i.astype(jnp.float32)
                x1f = x1i.astype(jnp.float32)
                y0f = y0i.astype(jnp.float32)
                y1f = y1i.astype(jnp.float32)
                idx[b][0][sl] = x0i * W + y0i
                idx[b][1][sl] = x0i * W + y1i
                idx[b][2][sl] = x1i * W + y0i
                idx[b][3][sl] = x1i * W + y1i
                wts[b][0][sl] = (x1f - xc) * (y1f - yc)
                wts[b][1][sl] = (x1f - xc) * (yc - y0f)
                wts[b][2][sl] = (xc - x0f) * (y1f - yc)
                wts[b][3][sl] = (xc - x0f) * (yc - y0f)
            for c in range(4):
                pltpu.async_copy(table_hbm.at[idx[b][c]], rows[b][c], sg[b][c])

        def wait_gathers(b):
            for c in range(4):
                pltpu.make_async_copy(
                    table_hbm.at[idx[b][c]], rows[b][c], sg[b][c]).wait()

        def combine(b):
            ar, br, cr, dr = rows[b]

            def grp_body(g, c2):
                gs = pl.ds(g * LANES, LANES)
                wa16 = wts[b][0][gs]
                wb16 = wts[b][1][gs]
                wc16 = wts[b][2][gs]
                wd16 = wts[b][3][gs]
                row0 = g * LANES
                for p in range(LANES):
                    wa = wa16[p]
                    wb = wb16[p]
                    wc = wc16[p]
                    wd = wd16[p]
                    r = row0 + p
                    for j in range(C // LANES):
                        sj = pl.ds(j * LANES, LANES)
                        ov[b][r, sj] = (wa * ar[r, sj] + wb * br[r, sj]
                                        + wc * cr[r, sj] + wd * dr[r, sj])
                return c2

            lax.fori_loop(0, P // LANES, grp_body, 0)

        def fire_out(k, b):
            off = base + k * P
            pltpu.async_copy(ov[b], out_hbm.at[pl.ds(off, P)], so[b])

        def wait_out(b):
            pltpu.make_async_copy(
                ov[b], out_hbm.at[pl.ds(base, P)], so[b]).wait()

        fire(0, 0)

        def pair_body(kk, carry):
            for bph in range(2):
                k = 2 * kk + bph

                @pl.when(k + 1 < NCHUNK)
                def _fire_next():
                    fire(k + 1, 1 - bph)

                wait_gathers(bph)

                @pl.when(k >= 2)
                def _drain_out():
                    wait_out(bph)

                combine(bph)
                fire_out(k, bph)
            return carry

        lax.fori_loop(0, NCHUNK // 2, pair_body, 0)
        wait_out(0)
        wait_out(1)

    return bilinear_sc


_bilinear_sc = _make_sc_kernel()


def kernel(images, coordinates):
    B, c, h, w = images.shape
    table = images.reshape(c, h * w).T
    xs = coordinates[:, 0].reshape(-1)
    ys = coordinates[:, 1].reshape(-1)
    out_nc = _bilinear_sc(table, xs, ys)
    new_h, new_w = coordinates.shape[2], coordinates.shape[3]
    return out_nc.T.reshape(B, c, new_h, new_w)


# speedup vs baseline: 1.1914x; 1.0708x over previous
import functools

import jax
import jax.numpy as jnp
from jax import lax
from jax.experimental import pallas as pl
from jax.experimental.pallas import tpu as pltpu
from jax.experimental.pallas import tpu_sc as plsc

H = 384
W = 384
C = 192
N = H * W
NC = 2
NS = 16
NW = NC * NS
LANES = 16
N_PER_W = N // NW
P = 48
NCHUNK = N_PER_W // P


def _scratch_types():
    per_set = (
        [pltpu.VMEM((P,), jnp.int32)] * 4
        + [pltpu.VMEM((P,), jnp.float32)] * 4
        + [pltpu.VMEM((P, C), jnp.float32)] * 4
        + [pltpu.VMEM((P, C), jnp.float32)]
    )
    return ([pltpu.VMEM((N_PER_W,), jnp.float32)] * 2
            + per_set * 2
            + [pltpu.SemaphoreType.DMA] * 8
            + [pltpu.SemaphoreType.DMA] * 2)


def _make_sc_kernel():
    mesh = plsc.VectorSubcoreMesh(core_axis_name="c", subcore_axis_name="s")

    @functools.partial(
        pl.kernel,
        mesh=mesh,
        compiler_params=pltpu.CompilerParams(use_tc_tiling_on_sc=False),
        out_type=jax.ShapeDtypeStruct((N, C), jnp.float32),
        scratch_types=_scratch_types(),
    )
    def bilinear_sc(table_hbm, xs_hbm, ys_hbm, out_hbm, *s):
        it = iter(s)
        xall = next(it)
        yall = next(it)
        idx, wts, rows, ov = [], [], [], []
        for _ in range(2):
            idx.append([next(it) for _ in range(4)])
            wts.append([next(it) for _ in range(4)])
            rows.append([next(it) for _ in range(4)])
            ov.append(next(it))
        sg = [[next(it) for _ in range(4)] for _ in range(2)]
        so = [next(it) for _ in range(2)]

        wid = lax.axis_index("s") * NC + lax.axis_index("c")
        base = wid * N_PER_W

        def fire(k, b):
            for i in range(P // LANES):
                sl = pl.ds(i * LANES, LANES)
                cs = pl.ds(k * P + i * LANES, LANES)
                x = xall[cs]
                y = yall[cs]
                x0i = jnp.minimum(jnp.maximum(x.astype(jnp.int32), 0), H - 1)
                y0i = jnp.minimum(jnp.maximum(y.astype(jnp.int32), 0), W - 1)
                x1i = jnp.minimum(x0i + 1, H - 1)
                y1i = jnp.minimum(y0i + 1, W - 1)
                xc = jnp.minimum(jnp.maximum(x, 0.0), float(H - 1))
                yc = jnp.minimum(jnp.maximum(y, 0.0), float(W - 1))
                x0f = x0i.astype(jnp.float32)
                x1f = x1i.astype(jnp.float32)
                y0f = y0i.astype(jnp.float32)
                y1f = y1i.astype(jnp.float32)
                idx[b][0][sl] = x0i * W + y0i
                idx[b][1][sl] = x0i * W + y1i
                idx[b][2][sl] = x1i * W + y0i
                idx[b][3][sl] = x1i * W + y1i
                wts[b][0][sl] = (x1f - xc) * (y1f - yc)
                wts[b][1][sl] = (x1f - xc) * (yc - y0f)
                wts[b][2][sl] = (xc - x0f) * (y1f - yc)
                wts[b][3][sl] = (xc - x0f) * (yc - y0f)
            for c in range(4):
                pltpu.async_copy(table_hbm.at[idx[b][c]], rows[b][c], sg[b][c])

        def wait_gathers(b):
            for c in range(4):
                pltpu.make_async_copy(
                    table_hbm.at[idx[b][c]], rows[b][c], sg[b][c]).wait()

        def combine(b):
            ar, br, cr, dr = rows[b]

            def grp_body(g, c2):
                gs = pl.ds(g * LANES, LANES)
                wa16 = wts[b][0][gs]
                wb16 = wts[b][1][gs]
                wc16 = wts[b][2][gs]
                wd16 = wts[b][3][gs]
                row0 = g * LANES
                for p in range(LANES):
                    wa = wa16[p]
                    wb = wb16[p]
                    wc = wc16[p]
                    wd = wd16[p]
                    r = row0 + p
                    for j in range(C // LANES):
                        sj = pl.ds(j * LANES, LANES)
                        ov[b][r, sj] = (wa * ar[r, sj] + wb * br[r, sj]
                                        + wc * cr[r, sj] + wd * dr[r, sj])
                return c2

            lax.fori_loop(0, P // LANES, grp_body, 0)

        def fire_out(k, b):
            off = base + k * P
            pltpu.async_copy(ov[b], out_hbm.at[pl.ds(off, P)], so[b])

        def wait_out(b):
            pltpu.make_async_copy(
                ov[b], out_hbm.at[pl.ds(base, P)], so[b]).wait()

        pltpu.sync_copy(xs_hbm.at[pl.ds(base, N_PER_W)], xall)
        pltpu.sync_copy(ys_hbm.at[pl.ds(base, N_PER_W)], yall)
        fire(0, 0)

        def pair_body(kk, carry):
            for bph in range(2):
                k = 2 * kk + bph

                @pl.when(k + 1 < NCHUNK)
                def _fire_next():
                    fire(k + 1, 1 - bph)

                wait_gathers(bph)

                @pl.when(k >= 2)
                def _drain_out():
                    wait_out(bph)

                combine(bph)
                fire_out(k, bph)
            return carry

        lax.fori_loop(0, NCHUNK // 2, pair_body, 0)
        wait_out(0)
        wait_out(1)

    return bilinear_sc


_bilinear_sc = _make_sc_kernel()


def kernel(images, coordinates):
    B, c, h, w = images.shape
    table = images.reshape(c, h * w).T
    xs = coordinates[:, 0].reshape(-1)
    ys = coordinates[:, 1].reshape(-1)
    out_nc = _bilinear_sc(table, xs, ys)
    new_h, new_w = coordinates.shape[2], coordinates.shape[3]
    return out_nc.T.reshape(B, c, new_h, new_w)
